# baseline (speedup 1.0000x reference)
import jax
import jax.numpy as jnp
from jax.experimental import pallas as pl
from jax.experimental.pallas import tpu as pltpu

B, T, D, DFF, H, DIN, NEXP = 1, 2048, 768, 2048, 12, 6, 6
DH = D // H
HP = H // 2
DH2 = 2 * DH
VW = 128
KC = 512
DFH = DFF // 2
TEMP = 1.5
EPS = 1e-6
F32 = jnp.float32
BF16 = jnp.bfloat16
I32 = jnp.int32
NEG = -1e9
BT = 256
NTB = T // BT

_SMEM_SPEC = pl.BlockSpec(memory_space=pltpu.SMEM)


def _rms_in(x, g):
    ms = jnp.mean(x * x, axis=-1, keepdims=True)
    return x * g * jax.lax.rsqrt(ms + EPS)


def _ffn_rows(y16, w1, b1, w2):
    h = (jnp.dot(y16, w1, preferred_element_type=F32) + b1).astype(BF16)
    return jnp.dot(jax.nn.gelu(h), w2, preferred_element_type=F32)


def _embed_kernel(meta_ref, xin_ref, w_ref, b_ref, o_ref):
    nb = meta_ref[1]
    w = w_ref[...]
    b = b_ref[...]

    def body(i, _):
        o_ref[pl.ds(i * BT, BT), :] = (
            jnp.dot(xin_ref[pl.ds(i * BT, BT), :], w,
                    preferred_element_type=F32) + b)
        return 0

    jax.lax.fori_loop(0, nb, body, 0)


def _embed(xin, meta, w_in, b_in):
    return pl.pallas_call(
        _embed_kernel,
        in_specs=[_SMEM_SPEC] + [pl.BlockSpec()] * 3,
        out_shape=jax.ShapeDtypeStruct((T, D), F32),
    )(meta, xin, w_in, b_in.reshape(1, D))


def _attn_phase(h2, meta_ref, x_ref, g_ref, wq_ref, wk_ref, wv_ref, wo_ref,
                o_ref, y_ref, k_ref, v_ref):
    kk = meta_ref[0]
    nb = meta_ref[1]

    @pl.when(h2 == 0)
    def _():
        xx = x_ref[...]
        y_ref[...] = _rms_in(xx, g_ref[...]).astype(BF16)
        o_ref[...] = xx

    wk = wk_ref[...].astype(BF16)
    wv = wv_ref[...].astype(BF16)
    v_ref[...] = jnp.zeros((T, 2 * VW), BF16)
    ones = jnp.ones((BT, 1), BF16)
    zpad = jnp.zeros((BT, VW - DH - 1), BF16)

    def kv_body(i, _):
        yb = y_ref[pl.ds(i * BT, BT), :]
        kb = jnp.dot(yb, wk, preferred_element_type=F32)
        vb = jnp.dot(yb, wv, preferred_element_type=F32)
        rid = i * BT + jax.lax.broadcasted_iota(I32, (BT, 1), 0)
        vb = jnp.where(rid < kk, vb, 0.0).astype(BF16)
        k_ref[pl.ds(i * BT, BT), :] = (kb * 0.125).astype(BF16)
        v_ref[pl.ds(i * BT, BT), :] = jnp.concatenate(
            [vb[:, :DH], ones, zpad, vb[:, DH:], ones, zpad], axis=1)
        return 0

    jax.lax.fori_loop(0, nb, kv_body, 0)

    wq = wq_ref[...].astype(BF16)
    wo = wo_ref[...].astype(BF16)
    colmask = jax.lax.broadcasted_iota(I32, (1, T), 1) < kk
    neg16 = jnp.asarray(NEG, BF16)
    kx = k_ref[...]
    vx = v_ref[...]

    def q_body(i, _):
        yb = y_ref[pl.ds(i * BT, BT), :]
        q2 = jnp.dot(yb, wq, preferred_element_type=F32).astype(BF16)
        obs = []
        for s in range(2):
            qb = q2[:, s * DH:(s + 1) * DH]
            ks = kx[:, s * DH:(s + 1) * DH]
            vs = vx[:, s * VW:(s + 1) * VW]
            att = jax.lax.dot_general(qb, ks, (((1,), (1,)), ((), ())),
                                      preferred_element_type=F32).astype(BF16)
            att = jnp.where(colmask, att, neg16)
            m = jnp.max(att, axis=-1, keepdims=True)
            p16 = jnp.exp(att - m)
            of = jnp.dot(p16, vs, preferred_element_type=F32)
            obs.append(of[:, :DH] / of[:, DH:DH + 1])
        o2 = jnp.concatenate(obs, axis=1).astype(BF16)
        o_ref[pl.ds(i * BT, BT), :] += jnp.dot(o2, wo,
                                               preferred_element_type=F32)
        return 0

    jax.lax.fori_loop(0, nb, q_body, 0)


def _bb_layer_kernel(meta_ref, x_ref, g1_ref, wq_ref, wk_ref, wv_ref, wo_ref,
                     g2_ref, w1_ref, b1_ref, w2_ref, b2_ref,
                     o_ref, y_ref, k_ref, v_ref):
    step = pl.program_id(0)
    nb = meta_ref[1]

    @pl.when(step < HP)
    def _():
        _attn_phase(step, meta_ref, x_ref, g1_ref, wq_ref, wk_ref, wv_ref,
                    wo_ref, o_ref, y_ref, k_ref, v_ref)

    @pl.when(step == HP)
    def _():
        g = g2_ref[...]
        w1 = w1_ref[...].astype(BF16)
        b1 = b1_ref[...]
        w2 = w2_ref[...].astype(BF16)
        b2 = b2_ref[...]

        def body(i, _):
            xb = o_ref[pl.ds(i * BT, BT), :]
            yb = _rms_in(xb, g).astype(BF16)
            o_ref[pl.ds(i * BT, BT), :] = xb + _ffn_rows(yb, w1, b1, w2) + b2
            return 0

        jax.lax.fori_loop(0, nb, body, 0)


def _bb_layer(x, meta, lp):
    full = pl.BlockSpec((T, D), lambda s: (0, 0))
    one = pl.BlockSpec((1, D), lambda s: (0, 0))
    hcol = lambda s: (0, jnp.minimum(s, HP - 1))
    return pl.pallas_call(
        _bb_layer_kernel,
        grid=(HP + 1,),
        in_specs=[
            _SMEM_SPEC,
            full,
            one,
            pl.BlockSpec((D, DH2), hcol),
            pl.BlockSpec((D, DH2), hcol),
            pl.BlockSpec((D, DH2), hcol),
            pl.BlockSpec((DH2, D), lambda s: (jnp.minimum(s, HP - 1), 0)),
            one,
            pl.BlockSpec((D, DFF), lambda s: (0, 0)),
            pl.BlockSpec((1, DFF), lambda s: (0, 0)),
            pl.BlockSpec((DFF, D), lambda s: (0, 0)),
            one,
        ],
        out_specs=full,
        out_shape=jax.ShapeDtypeStruct((T, D), F32),
        scratch_shapes=[pltpu.VMEM((T, D), BF16),
                        pltpu.VMEM((T, DH2), BF16),
                        pltpu.VMEM((T, 2 * VW), BF16)],
    )(meta, x, lp['ln1_g'].reshape(1, D), lp['wq'], lp['wk'], lp['wv'],
      lp['wo'], lp['ln2_g'].reshape(1, D), lp['w1'], lp['b1'].reshape(1, DFF),
      lp['w2'], lp['b2'].reshape(1, D))


def _moe_layer_kernel(meta_ref, x_ref, g1_ref, wq_ref, wk_ref, wv_ref, wo_ref,
                      g2_ref, w1s_ref, b1s_ref, w2s_ref, b2s_ref,
                      w1e_ref, b1e_ref, w2e_ref, b2e_ref, c_ref,
                      o_ref, y_ref, k_ref, v_ref):
    step = pl.program_id(0)
    nb = meta_ref[1]

    @pl.when(step < HP)
    def _():
        _attn_phase(step, meta_ref, x_ref, g1_ref, wq_ref, wk_ref, wv_ref,
                    wo_ref, o_ref, y_ref, k_ref, v_ref)

    @pl.when(step >= HP)
    def _():
        half = (step - HP) % 2

        @pl.when(step == HP)
        def _():
            g = g2_ref[...]
            w1s = w1s_ref[...]
            b1s = b1s_ref[...]
            w2s = w2s_ref[...]
            b2s = b2s_ref[...]

            def sbody(i, _):
                xb = o_ref[pl.ds(i * BT, BT), :]
                yb = _rms_in(xb, g).astype(BF16)
                y_ref[pl.ds(i * BT, BT), :] = yb
                o_ref[pl.ds(i * BT, BT), :] = (xb + _ffn_rows(yb, w1s, b1s, w2s)
                                               + b2s)
                return 0

            jax.lax.fori_loop(0, nb, sbody, 0)

        c = c_ref[0, 0, 0]
        w1e = w1e_ref[0].astype(BF16)
        b1e = b1e_ref[0]
        w2e = w2e_ref[0].astype(BF16)
        b2eff = jnp.where(half == 0, b2e_ref[0], 0.0)

        def ebody(i, _):
            yb = y_ref[pl.ds(i * BT, BT), :]
            o_ref[pl.ds(i * BT, BT), :] += c * (_ffn_rows(yb, w1e, b1e, w2e)
                                                + b2eff)
            return 0

        jax.lax.fori_loop(0, nb, ebody, 0)


def _moe_layer(x, meta, w, lp):
    ew = lp['experts']
    sh = lp['shared']
    full = pl.BlockSpec((T, D), lambda s: (0, 0))
    one = pl.BlockSpec((1, D), lambda s: (0, 0))
    hcol = lambda s: (0, jnp.minimum(s, HP - 1))
    eix = lambda s: jnp.clip((s - HP) // 2, 0, NEXP - 1)
    hix = lambda s: jnp.clip((s - HP) % 2, 0, 1)
    return pl.pallas_call(
        _moe_layer_kernel,
        grid=(HP + 2 * NEXP,),
        in_specs=[
            _SMEM_SPEC,
            full,
            one,
            pl.BlockSpec((D, DH2), hcol),
            pl.BlockSpec((D, DH2), hcol),
            pl.BlockSpec((D, DH2), hcol),
            pl.BlockSpec((DH2, D), lambda s: (jnp.minimum(s, HP - 1), 0)),
            one,
            pl.BlockSpec((D, DFF), lambda s: (0, 0)),
            pl.BlockSpec((1, DFF), lambda s: (0, 0)),
            pl.BlockSpec((DFF, D), lambda s: (0, 0)),
            one,
            pl.BlockSpec((1, D, DFH), lambda s: (eix(s), 0, hix(s))),
            pl.BlockSpec((1, 1, DFH), lambda s: (eix(s), 0, hix(s))),
            pl.BlockSpec((1, DFH, D), lambda s: (eix(s), hix(s), 0)),
            pl.BlockSpec((1, 1, D), lambda s: (eix(s), 0, 0)),
            pl.BlockSpec((1, 1, 1), lambda s: (eix(s), 0, 0)),
        ],
        out_specs=full,
        out_shape=jax.ShapeDtypeStruct((T, D), F32),
        scratch_shapes=[pltpu.VMEM((T, D), BF16),
                        pltpu.VMEM((T, DH2), BF16),
                        pltpu.VMEM((T, 2 * VW), BF16)],
    )(meta, x, lp['ln1_g'].reshape(1, D), lp['wq'], lp['wk'], lp['wv'],
      lp['wo'], lp['ln2_g'].reshape(1, D),
      sh['w1'].astype(BF16), sh['b1'].reshape(1, DFF),
      sh['w2'].astype(BF16), sh['b2'].reshape(1, D),
      ew['w1'], ew['b1'].reshape(NEXP, 1, DFF),
      ew['w2'], ew['b2'].reshape(NEXP, 1, D),
      w.reshape(NEXP, 1, 1))


def _router_kernel(meta_ref, tok_ref, crd_ref, mk_ref, w1a_ref, w1b_ref,
                   b1_ref, w2_ref, b2_ref, lg_ref, w_ref):
    kk = meta_ref[0]
    rowmask = jax.lax.broadcasted_iota(I32, (T, 1), 0) < kk
    cnt = jnp.maximum(kk.astype(F32), 1.0)
    pr = jnp.sum(jnp.where(rowmask, tok_ref[...], 0.0), axis=0,
                 keepdims=True) / cnt
    ct = jnp.sum(crd_ref[...] * mk_ref[...], axis=0, keepdims=True) / cnt
    hr = jax.nn.gelu(jnp.dot(pr, w1a_ref[...], preferred_element_type=F32)
                     + jnp.dot(ct, w1b_ref[...], preferred_element_type=F32)
                     + b1_ref[...])
    lg = jnp.dot(hr, w2_ref[...], preferred_element_type=F32) + b2_ref[...]
    lg_ref[...] = lg
    z = lg / TEMP
    z = z - jnp.max(z, axis=-1, keepdims=True)
    p = jnp.exp(z)
    w_ref[...] = jnp.maximum(p / jnp.sum(p, axis=-1, keepdims=True), 1e-6)


def _router(tokens, meta, coords2, mkcol, rp):
    return pl.pallas_call(
        _router_kernel,
        in_specs=[_SMEM_SPEC] + [pl.BlockSpec()] * 8,
        out_shape=(jax.ShapeDtypeStruct((1, NEXP), F32),
                   jax.ShapeDtypeStruct((1, NEXP), F32)),
    )(meta, tokens, coords2, mkcol, rp['w1'][:D], rp['w1'][D:],
      rp['b1'].reshape(1, D), rp['w2'], rp['b2'].reshape(1, NEXP))


def _final_kernel(meta_ref, x_ref, g_ref, ew1_ref, eb1_ref, ew2_ref, eb2_ref,
                  dw1_ref, db1_ref, dw2_ref, db2_ref, en_ref, dr_ref):
    kk = meta_ref[0]
    nb = meta_ref[1]
    g = g_ref[...]
    cnt = jnp.maximum(kk.astype(F32), 1.0)

    def body(i, acc):
        xb = _rms_in(x_ref[pl.ds(i * BT, BT), :], g)
        rid = i * BT + jax.lax.broadcasted_iota(I32, (BT, 1), 0)
        return acc + jnp.sum(jnp.where(rid < kk, xb, 0.0), axis=0,
                             keepdims=True)

    p = jax.lax.fori_loop(0, nb, body, jnp.zeros((1, D), F32)) / cnt
    he = jax.nn.gelu(jnp.dot(p, ew1_ref[...], preferred_element_type=F32)
                     + eb1_ref[...])
    en_ref[...] = jnp.dot(he, ew2_ref[...], preferred_element_type=F32) + eb2_ref[...]
    hd = jax.nn.gelu(jnp.dot(p, dw1_ref[...], preferred_element_type=F32)
                     + db1_ref[...])
    dr_ref[...] = jnp.dot(hd, dw2_ref[...], preferred_element_type=F32) + db2_ref[...]


def _final(x, meta, params):
    eh, dh = params['eh'], params['dh']
    return pl.pallas_call(
        _final_kernel,
        in_specs=[_SMEM_SPEC] + [pl.BlockSpec()] * 10,
        out_shape=(jax.ShapeDtypeStruct((1, 2), F32),
                   jax.ShapeDtypeStruct((1, 3), F32)),
    )(meta, x, params['final_g'].reshape(1, D), eh['w1'],
      eh['b1'].reshape(1, D), eh['w2'], eh['b2'].reshape(1, 2),
      dh['w1'], dh['b1'].reshape(1, D), dh['w2'], dh['b2'].reshape(1, 3))


def kernel(coords, features, batch_ids, params):
    c2 = coords[0]
    mask = batch_ids[0] > 0
    mkcol = mask.astype(F32).reshape(T, 1)
    kcount = jnp.sum(mask.astype(I32))
    nblk = (kcount + BT - 1) // BT
    meta = jnp.stack([kcount, nblk])
    perm = jnp.argsort(jnp.logical_not(mask))
    xin = jnp.concatenate([c2, features[0]], axis=-1)[perm]
    x = _embed(xin, meta, params['w_in'], params['b_in'])
    for lp in params['backbone']:
        x = _bb_layer(x, meta, lp)
    lg, w = _router(x, meta, c2, mkcol, params['router'])
    for lp in params['moe']:
        x = _moe_layer(x, meta, w[0], lp)
    en, dr = _final(x, meta, params)
    return jnp.concatenate([lg, en, dr], axis=-1)


# speedup vs baseline: 1.0036x; 1.0036x over previous
import jax
import jax.numpy as jnp
from jax.experimental import pallas as pl
from jax.experimental.pallas import tpu as pltpu

B, T, D, DFF, H, DIN, NEXP = 1, 2048, 768, 2048, 12, 6, 6
DH = D // H
HP = H // 2
DH2 = 2 * DH
VW = 128
DFH = DFF // 2
TEMP = 1.5
EPS = 1e-6
F32 = jnp.float32
BF16 = jnp.bfloat16
I32 = jnp.int32
NEG = -1e9
BT = 256

_SMEM_SPEC = pl.BlockSpec(memory_space=pltpu.SMEM)


def _rms_in(x, g):
    ms = jnp.mean(x * x, axis=-1, keepdims=True)
    return x * g * jax.lax.rsqrt(ms + EPS)


def _ffn_rows(y16, w1, b1, w2):
    h = (jnp.dot(y16, w1, preferred_element_type=F32) + b1).astype(BF16)
    return jnp.dot(jax.nn.gelu(h), w2, preferred_element_type=F32)


def _embed_kernel(xin_ref, w_ref, b_ref, o_ref):
    o_ref[...] = (jnp.dot(xin_ref[...], w_ref[...], preferred_element_type=F32)
                  + b_ref[...])


def _embed(xin, w_in, b_in):
    return pl.pallas_call(
        _embed_kernel,
        out_shape=jax.ShapeDtypeStruct((T, D), F32),
    )(xin, w_in, b_in.reshape(1, D))


def _attn_phase(h2, meta_ref, x_ref, g_ref, wq_ref, wk_ref, wv_ref, wo_ref,
                o_ref, y_ref, k_ref, v_ref):
    kk = meta_ref[0]
    nb = meta_ref[1]

    @pl.when(h2 == 0)
    def _():
        xx = x_ref[...]
        y_ref[...] = _rms_in(xx, g_ref[...]).astype(BF16)
        o_ref[...] = xx

    wk = wk_ref[...].astype(BF16)
    wv = wv_ref[...].astype(BF16)
    v_ref[...] = jnp.zeros((T, 2 * VW), BF16)
    ones = jnp.ones((BT, 1), BF16)
    zpad = jnp.zeros((BT, VW - DH - 1), BF16)

    def kv_body(i, _):
        yb = y_ref[pl.ds(i * BT, BT), :]
        kb = jnp.dot(yb, wk, preferred_element_type=F32)
        vb = jnp.dot(yb, wv, preferred_element_type=F32)
        rid = i * BT + jax.lax.broadcasted_iota(I32, (BT, 1), 0)
        vb = jnp.where(rid < kk, vb, 0.0).astype(BF16)
        k_ref[pl.ds(i * BT, BT), :] = (kb * 0.125).astype(BF16)
        v_ref[pl.ds(i * BT, BT), :] = jnp.concatenate(
            [vb[:, :DH], ones, zpad, vb[:, DH:], ones, zpad], axis=1)
        return 0

    jax.lax.fori_loop(0, nb, kv_body, 0)

    wq = wq_ref[...].astype(BF16)
    wo = wo_ref[...].astype(BF16)
    colmask = jax.lax.broadcasted_iota(I32, (1, T), 1) < kk
    neg16 = jnp.asarray(NEG, BF16)
    kx = k_ref[...]
    vx = v_ref[...]

    def q_body(i, _):
        yb = y_ref[pl.ds(i * BT, BT), :]
        q2 = jnp.dot(yb, wq, preferred_element_type=F32).astype(BF16)
        obs = []
        for s in range(2):
            qb = q2[:, s * DH:(s + 1) * DH]
            ks = kx[:, s * DH:(s + 1) * DH]
            vs = vx[:, s * VW:(s + 1) * VW]
            att = jax.lax.dot_general(qb, ks, (((1,), (1,)), ((), ())),
                                      preferred_element_type=F32).astype(BF16)
            att = jnp.where(colmask, att, neg16)
            m = jnp.max(att, axis=-1, keepdims=True)
            p16 = jnp.exp(att - m)
            of = jnp.dot(p16, vs, preferred_element_type=F32)
            obs.append(of[:, :DH] / of[:, DH:DH + 1])
        o2 = jnp.concatenate(obs, axis=1).astype(BF16)
        o_ref[pl.ds(i * BT, BT), :] += jnp.dot(o2, wo,
                                               preferred_element_type=F32)
        return 0

    jax.lax.fori_loop(0, nb, q_body, 0)


def _bb_layer_kernel(meta_ref, x_ref, g1_ref, wq_ref, wk_ref, wv_ref, wo_ref,
                     g2_ref, w1_ref, b1_ref, w2_ref, b2_ref,
                     o_ref, y_ref, k_ref, v_ref):
    step = pl.program_id(0)
    nb = meta_ref[1]

    @pl.when(step < HP)
    def _():
        _attn_phase(step, meta_ref, x_ref, g1_ref, wq_ref, wk_ref, wv_ref,
                    wo_ref, o_ref, y_ref, k_ref, v_ref)

    @pl.when(step == HP)
    def _():
        g = g2_ref[...]
        w1 = w1_ref[...].astype(BF16)
        b1 = b1_ref[...]
        w2 = w2_ref[...].astype(BF16)
        b2 = b2_ref[...]

        def body(i, _):
            xb = o_ref[pl.ds(i * BT, BT), :]
            yb = _rms_in(xb, g).astype(BF16)
            o_ref[pl.ds(i * BT, BT), :] = xb + _ffn_rows(yb, w1, b1, w2) + b2
            return 0

        jax.lax.fori_loop(0, nb, body, 0)


def _bb_layer(x, meta, lp):
    full = pl.BlockSpec((T, D), lambda s: (0, 0))
    one = pl.BlockSpec((1, D), lambda s: (0, 0))
    hcol = lambda s: (0, jnp.minimum(s, HP - 1))
    return pl.pallas_call(
        _bb_layer_kernel,
        grid=(HP + 1,),
        in_specs=[
            _SMEM_SPEC,
            full,
            one,
            pl.BlockSpec((D, DH2), hcol),
            pl.BlockSpec((D, DH2), hcol),
            pl.BlockSpec((D, DH2), hcol),
            pl.BlockSpec((DH2, D), lambda s: (jnp.minimum(s, HP - 1), 0)),
            one,
            pl.BlockSpec((D, DFF), lambda s: (0, 0)),
            pl.BlockSpec((1, DFF), lambda s: (0, 0)),
            pl.BlockSpec((DFF, D), lambda s: (0, 0)),
            one,
        ],
        out_specs=full,
        out_shape=jax.ShapeDtypeStruct((T, D), F32),
        scratch_shapes=[pltpu.VMEM((T, D), BF16),
                        pltpu.VMEM((T, DH2), BF16),
                        pltpu.VMEM((T, 2 * VW), BF16)],
    )(meta, x, lp['ln1_g'].reshape(1, D), lp['wq'], lp['wk'], lp['wv'],
      lp['wo'], lp['ln2_g'].reshape(1, D), lp['w1'], lp['b1'].reshape(1, DFF),
      lp['w2'], lp['b2'].reshape(1, D))


def _moe_layer_kernel(meta_ref, x_ref, g1_ref, wq_ref, wk_ref, wv_ref, wo_ref,
                      g2_ref, w1s_ref, b1s_ref, w2s_ref, b2s_ref,
                      w1e_ref, b1e_ref, w2e_ref, b2e_ref, c_ref,
                      o_ref, y_ref, k_ref, v_ref):
    step = pl.program_id(0)
    nb = meta_ref[1]

    @pl.when(step < HP)
    def _():
        _attn_phase(step, meta_ref, x_ref, g1_ref, wq_ref, wk_ref, wv_ref,
                    wo_ref, o_ref, y_ref, k_ref, v_ref)

    @pl.when(step >= HP)
    def _():
        half = (step - HP) % 2

        @pl.when(step == HP)
        def _():
            g = g2_ref[...]
            w1s = w1s_ref[...]
            b1s = b1s_ref[...]
            w2s = w2s_ref[...]
            b2s = b2s_ref[...]

            def sbody(i, _):
                xb = o_ref[pl.ds(i * BT, BT), :]
                yb = _rms_in(xb, g).astype(BF16)
                y_ref[pl.ds(i * BT, BT), :] = yb
                o_ref[pl.ds(i * BT, BT), :] = (xb + _ffn_rows(yb, w1s, b1s, w2s)
                                               + b2s)
                return 0

            jax.lax.fori_loop(0, nb, sbody, 0)

        c = c_ref[0, 0, 0]
        w1e = w1e_ref[0].astype(BF16)
        b1e = b1e_ref[0]
        w2e = w2e_ref[0].astype(BF16)
        b2eff = jnp.where(half == 0, b2e_ref[0], 0.0)

        def ebody(i, _):
            yb = y_ref[pl.ds(i * BT, BT), :]
            o_ref[pl.ds(i * BT, BT), :] += c * (_ffn_rows(yb, w1e, b1e, w2e)
                                                + b2eff)
            return 0

        jax.lax.fori_loop(0, nb, ebody, 0)


def _moe_layer(x, meta, w, lp):
    ew = lp['experts']
    sh = lp['shared']
    full = pl.BlockSpec((T, D), lambda s: (0, 0))
    one = pl.BlockSpec((1, D), lambda s: (0, 0))
    hcol = lambda s: (0, jnp.minimum(s, HP - 1))
    eix = lambda s: jnp.clip((s - HP) // 2, 0, NEXP - 1)
    hix = lambda s: jnp.clip((s - HP) % 2, 0, 1)
    return pl.pallas_call(
        _moe_layer_kernel,
        grid=(HP + 2 * NEXP,),
        in_specs=[
            _SMEM_SPEC,
            full,
            one,
            pl.BlockSpec((D, DH2), hcol),
            pl.BlockSpec((D, DH2), hcol),
            pl.BlockSpec((D, DH2), hcol),
            pl.BlockSpec((DH2, D), lambda s: (jnp.minimum(s, HP - 1), 0)),
            one,
            pl.BlockSpec((D, DFF), lambda s: (0, 0)),
            pl.BlockSpec((1, DFF), lambda s: (0, 0)),
            pl.BlockSpec((DFF, D), lambda s: (0, 0)),
            one,
            pl.BlockSpec((1, D, DFH), lambda s: (eix(s), 0, hix(s))),
            pl.BlockSpec((1, 1, DFH), lambda s: (eix(s), 0, hix(s))),
            pl.BlockSpec((1, DFH, D), lambda s: (eix(s), hix(s), 0)),
            pl.BlockSpec((1, 1, D), lambda s: (eix(s), 0, 0)),
            pl.BlockSpec((1, 1, 1), lambda s: (eix(s), 0, 0)),
        ],
        out_specs=full,
        out_shape=jax.ShapeDtypeStruct((T, D), F32),
        scratch_shapes=[pltpu.VMEM((T, D), BF16),
                        pltpu.VMEM((T, DH2), BF16),
                        pltpu.VMEM((T, 2 * VW), BF16)],
    )(meta, x, lp['ln1_g'].reshape(1, D), lp['wq'], lp['wk'], lp['wv'],
      lp['wo'], lp['ln2_g'].reshape(1, D),
      sh['w1'].astype(BF16), sh['b1'].reshape(1, DFF),
      sh['w2'].astype(BF16), sh['b2'].reshape(1, D),
      ew['w1'], ew['b1'].reshape(NEXP, 1, DFF),
      ew['w2'], ew['b2'].reshape(NEXP, 1, D),
      w.reshape(NEXP, 1, 1))


def _router_kernel(meta_ref, tok_ref, crd_ref, mk_ref, w1a_ref, w1b_ref,
                   b1_ref, w2_ref, b2_ref, lg_ref, w_ref):
    kk = meta_ref[0]
    rowmask = jax.lax.broadcasted_iota(I32, (T, 1), 0) < kk
    cnt = jnp.maximum(kk.astype(F32), 1.0)
    pr = jnp.sum(jnp.where(rowmask, tok_ref[...], 0.0), axis=0,
                 keepdims=True) / cnt
    ct = jnp.sum(crd_ref[...] * mk_ref[...], axis=0, keepdims=True) / cnt
    hr = jax.nn.gelu(jnp.dot(pr, w1a_ref[...], preferred_element_type=F32)
                     + jnp.dot(ct, w1b_ref[...], preferred_element_type=F32)
                     + b1_ref[...])
    lg = jnp.dot(hr, w2_ref[...], preferred_element_type=F32) + b2_ref[...]
    lg_ref[...] = lg
    z = lg / TEMP
    z = z - jnp.max(z, axis=-1, keepdims=True)
    p = jnp.exp(z)
    w_ref[...] = jnp.maximum(p / jnp.sum(p, axis=-1, keepdims=True), 1e-6)


def _router(tokens, meta, coords2, mkcol, rp):
    return pl.pallas_call(
        _router_kernel,
        in_specs=[_SMEM_SPEC] + [pl.BlockSpec()] * 8,
        out_shape=(jax.ShapeDtypeStruct((1, NEXP), F32),
                   jax.ShapeDtypeStruct((1, NEXP), F32)),
    )(meta, tokens, coords2, mkcol, rp['w1'][:D], rp['w1'][D:],
      rp['b1'].reshape(1, D), rp['w2'], rp['b2'].reshape(1, NEXP))


def _final_kernel(meta_ref, x_ref, g_ref, ew1_ref, eb1_ref, ew2_ref, eb2_ref,
                  dw1_ref, db1_ref, dw2_ref, db2_ref, en_ref, dr_ref):
    kk = meta_ref[0]
    xx = _rms_in(x_ref[...], g_ref[...])
    rowmask = jax.lax.broadcasted_iota(I32, (T, 1), 0) < kk
    cnt = jnp.maximum(kk.astype(F32), 1.0)
    p = jnp.sum(jnp.where(rowmask, xx, 0.0), axis=0, keepdims=True) / cnt
    he = jax.nn.gelu(jnp.dot(p, ew1_ref[...], preferred_element_type=F32)
                     + eb1_ref[...])
    en_ref[...] = jnp.dot(he, ew2_ref[...], preferred_element_type=F32) + eb2_ref[...]
    hd = jax.nn.gelu(jnp.dot(p, dw1_ref[...], preferred_element_type=F32)
                     + db1_ref[...])
    dr_ref[...] = jnp.dot(hd, dw2_ref[...], preferred_element_type=F32) + db2_ref[...]


def _final(x, meta, params):
    eh, dh = params['eh'], params['dh']
    return pl.pallas_call(
        _final_kernel,
        in_specs=[_SMEM_SPEC] + [pl.BlockSpec()] * 10,
        out_shape=(jax.ShapeDtypeStruct((1, 2), F32),
                   jax.ShapeDtypeStruct((1, 3), F32)),
    )(meta, x, params['final_g'].reshape(1, D), eh['w1'],
      eh['b1'].reshape(1, D), eh['w2'], eh['b2'].reshape(1, 2),
      dh['w1'], dh['b1'].reshape(1, D), dh['w2'], dh['b2'].reshape(1, 3))


def kernel(coords, features, batch_ids, params):
    c2 = coords[0]
    mask = batch_ids[0] > 0
    mkcol = mask.astype(F32).reshape(T, 1)
    kcount = jnp.sum(mask.astype(I32))
    nblk = (kcount + BT - 1) // BT
    meta = jnp.stack([kcount, nblk])
    perm = jnp.argsort(jnp.logical_not(mask))
    xin = jnp.concatenate([c2, features[0]], axis=-1)[perm]
    x = _embed(xin, params['w_in'], params['b_in'])
    for lp in params['backbone']:
        x = _bb_layer(x, meta, lp)
    lg, w = _router(x, meta, c2, mkcol, params['router'])
    for lp in params['moe']:
        x = _moe_layer(x, meta, w[0], lp)
    en, dr = _final(x, meta, params)
    return jnp.concatenate([lg, en, dr], axis=-1)
